# baseline (speedup 1.0000x reference)
import functools

import jax
import jax.numpy as jnp
from jax import lax
from jax.experimental import pallas as pl
from jax.experimental.pallas import tpu as pltpu
from jax.experimental.pallas import tpu_sc as plsc

N = 10000
E = 320000
IN_F = 128
H_F = 256
C_F = 64

NC = 2
NS = 16
NW = NC * NS
E_TILE = E // NW
CHUNK = 100
NCHUNK = E_TILE // CHUNK
N_PAD = 10240
N_TILE = N_PAD // NS

_MESH = plsc.VectorSubcoreMesh(
    core_axis_name="c", subcore_axis_name="s", num_cores=NC, num_subcores=NS
)


def _make_deg():
  @functools.partial(
      pl.kernel,
      out_type=jax.ShapeDtypeStruct((NC, N_PAD, 16), jnp.float32),
      mesh=_MESH,
      compiler_params=pltpu.CompilerParams(use_tc_tiling_on_sc=False),
      scratch_types=[
          pltpu.VMEM_SHARED((N_PAD, 16), jnp.float32),
          pltpu.VMEM((NCHUNK, CHUNK), jnp.int32),
          pltpu.VMEM((CHUNK, 16), jnp.float32),
      ],
  )
  def deg_kernel(ei_hbm, ones_hbm, zeros_hbm, out_hbm, acc, dst_v, ones_v):
    c = lax.axis_index("c")
    s = lax.axis_index("s")
    tid = c * NS + s
    pltpu.sync_copy(ei_hbm.at[1, tid], dst_v)
    pltpu.sync_copy(zeros_hbm, acc.at[pl.ds(s * N_TILE, N_TILE)])
    pltpu.sync_copy(ones_hbm, ones_v)
    plsc.subcore_barrier()

    def body(j, carry):
      pltpu.sync_copy(ones_v, acc.at[dst_v.at[j]], add=True)
      return carry

    lax.fori_loop(0, NCHUNK, body, 0)
    plsc.subcore_barrier()
    pltpu.sync_copy(
        acc.at[pl.ds(s * N_TILE, N_TILE)],
        out_hbm.at[c, pl.ds(s * N_TILE, N_TILE)],
    )

  return deg_kernel


def _make_prop(F, NBUF):
  @functools.partial(
      pl.kernel,
      out_type=jax.ShapeDtypeStruct((NC, N_PAD, F), jnp.float32),
      mesh=_MESH,
      compiler_params=pltpu.CompilerParams(use_tc_tiling_on_sc=False),
      scratch_types=[
          pltpu.VMEM_SHARED((N_PAD, F), jnp.float32),
          pltpu.VMEM((NCHUNK, CHUNK), jnp.int32),
          pltpu.VMEM((NCHUNK, CHUNK), jnp.int32),
      ]
      + [pltpu.VMEM((CHUNK, F), jnp.float32) for _ in range(NBUF)]
      + [pltpu.SemaphoreType.DMA for _ in range(NBUF)],
  )
  def prop_kernel(ei_hbm, xn_hbm, zeros_hbm, out_hbm, acc, src_v, dst_v, *bufs):
    rows = bufs[:NBUF]
    sems = bufs[NBUF:]
    c = lax.axis_index("c")
    s = lax.axis_index("s")
    tid = c * NS + s
    pltpu.sync_copy(ei_hbm.at[0, tid], src_v)
    pltpu.sync_copy(ei_hbm.at[1, tid], dst_v)
    pltpu.sync_copy(zeros_hbm, acc.at[pl.ds(s * N_TILE, N_TILE)])
    plsc.subcore_barrier()

    def stage(j, b):
      pltpu.async_copy(xn_hbm.at[src_v.at[j]], rows[b], sems[b])

    def finish(j, b):
      pltpu.make_async_copy(xn_hbm.at[src_v.at[j]], rows[b], sems[b]).wait()
      pltpu.sync_copy(rows[b], acc.at[dst_v.at[j]], add=True)

    for b in range(NBUF):
      stage(b, b)

    def body(g, carry):
      base = g * NBUF
      for b in range(NBUF):
        finish(base + b, b)
        stage(base + NBUF + b, b)
      return carry

    lax.fori_loop(0, NCHUNK // NBUF - 1, body, 0)
    for b in range(NBUF):
      finish(NCHUNK - NBUF + b, b)
    plsc.subcore_barrier()
    pltpu.sync_copy(
        acc.at[pl.ds(s * N_TILE, N_TILE)],
        out_hbm.at[c, pl.ds(s * N_TILE, N_TILE)],
    )

  return prop_kernel


_deg = _make_deg()
_prop128 = _make_prop(IN_F, 2)
_prop64 = _make_prop(C_F, 4)

BR = 1000
GRID = N // BR


def _dinv_block(dacc_ref):
  i = pl.program_id(0)
  rows = pl.ds(i * BR, BR)
  d = dacc_ref[0, rows, 0:1] + dacc_ref[1, rows, 0:1] + 1.0
  return lax.rsqrt(d)


_DACC_SPEC = pl.BlockSpec((NC, N_PAD, 16), lambda i: (0, 0, 0))


def _tc_scale(dacc, x):
  def body(dacc_ref, x_ref, o_ref):
    o_ref[...] = x_ref[...] * _dinv_block(dacc_ref)

  return pl.pallas_call(
      body,
      grid=(GRID,),
      in_specs=[
          _DACC_SPEC,
          pl.BlockSpec((BR, IN_F), lambda i: (i, 0)),
      ],
      out_specs=pl.BlockSpec((BR, IN_F), lambda i: (i, 0)),
      out_shape=jax.ShapeDtypeStruct((N, IN_F), jnp.float32),
  )(dacc, x)


def _tc_mid(dacc, agg1, xn1, W1, b1, W2):
  def body(dacc_ref, a_ref, xn_ref, W1_ref, b1_ref, W2_ref, o_ref):
    dinv = _dinv_block(dacc_ref)
    p = (a_ref[0] + a_ref[1] + xn_ref[...]) * dinv
    h = jnp.tanh(
        jnp.dot(p, W1_ref[...], preferred_element_type=jnp.float32)
        + b1_ref[...]
    )
    o_ref[...] = (
        jnp.dot(h, W2_ref[...], preferred_element_type=jnp.float32) * dinv
    )

  return pl.pallas_call(
      body,
      grid=(GRID,),
      in_specs=[
          _DACC_SPEC,
          pl.BlockSpec((NC, BR, IN_F), lambda i: (0, i, 0)),
          pl.BlockSpec((BR, IN_F), lambda i: (i, 0)),
          pl.BlockSpec((IN_F, H_F), lambda i: (0, 0)),
          pl.BlockSpec((1, H_F), lambda i: (0, 0)),
          pl.BlockSpec((H_F, C_F), lambda i: (0, 0)),
      ],
      out_specs=pl.BlockSpec((BR, C_F), lambda i: (i, 0)),
      out_shape=jax.ShapeDtypeStruct((N, C_F), jnp.float32),
  )(dacc, agg1, xn1, W1, b1, W2)


def _tc_final(dacc, agg2, xn2, b2):
  def body(dacc_ref, a_ref, xn_ref, b2_ref, o_ref):
    dinv = _dinv_block(dacc_ref)
    o_ref[...] = (a_ref[0] + a_ref[1] + xn_ref[...]) * dinv + b2_ref[...]

  return pl.pallas_call(
      body,
      grid=(GRID,),
      in_specs=[
          _DACC_SPEC,
          pl.BlockSpec((NC, BR, C_F), lambda i: (0, i, 0)),
          pl.BlockSpec((BR, C_F), lambda i: (i, 0)),
          pl.BlockSpec((1, C_F), lambda i: (0, 0)),
      ],
      out_specs=pl.BlockSpec((BR, C_F), lambda i: (i, 0)),
      out_shape=jax.ShapeDtypeStruct((N, C_F), jnp.float32),
  )(dacc, agg2, xn2, b2)


def kernel(edge_index, x, W1, b1, W2, b2):
  ei = edge_index.reshape(2, NW, NCHUNK, CHUNK)
  ones16 = jnp.ones((CHUNK, 16), jnp.float32)
  zdeg = jnp.zeros((N_TILE, 16), jnp.float32)
  z128 = jnp.zeros((N_TILE, IN_F), jnp.float32)
  z64 = jnp.zeros((N_TILE, C_F), jnp.float32)

  dacc = _deg(ei, ones16, zdeg)
  xn1 = _tc_scale(dacc, x)
  agg1 = _prop128(ei, xn1, z128)
  xn2 = _tc_mid(dacc, agg1, xn1, W1, b1.reshape(1, H_F), W2)
  agg2 = _prop64(ei, xn2, z64)
  return _tc_final(dacc, agg2, xn2, b2.reshape(1, C_F))


# speedup vs baseline: 1.0871x; 1.0871x over previous
import functools

import jax
import jax.numpy as jnp
from jax import lax
from jax.experimental import pallas as pl
from jax.experimental.pallas import tpu as pltpu
from jax.experimental.pallas import tpu_sc as plsc

N = 10000
E = 320000
IN_F = 128
H_F = 256
C_F = 64

NC = 2
NS = 16
NW = NC * NS
E_TILE = E // NW
CHUNK = 100
NCHUNK = E_TILE // CHUNK
N_PAD = 10240
N_TILE = N_PAD // NS

_MESH = plsc.VectorSubcoreMesh(
    core_axis_name="c", subcore_axis_name="s", num_cores=NC, num_subcores=NS
)


def _make_deg():
  @functools.partial(
      pl.kernel,
      out_type=jax.ShapeDtypeStruct((NC, N_PAD, 16), jnp.float32),
      mesh=_MESH,
      compiler_params=pltpu.CompilerParams(use_tc_tiling_on_sc=False),
      scratch_types=[
          pltpu.VMEM_SHARED((N_PAD, 16), jnp.float32),
          pltpu.VMEM((NCHUNK, CHUNK), jnp.int32),
          pltpu.VMEM((CHUNK, 16), jnp.float32),
      ],
  )
  def deg_kernel(ei_hbm, ones_hbm, zeros_hbm, out_hbm, acc, dst_v, ones_v):
    c = lax.axis_index("c")
    s = lax.axis_index("s")
    tid = c * NS + s
    pltpu.sync_copy(ei_hbm.at[1, tid], dst_v)
    pltpu.sync_copy(zeros_hbm, acc.at[pl.ds(s * N_TILE, N_TILE)])
    pltpu.sync_copy(ones_hbm, ones_v)
    plsc.subcore_barrier()

    def body(j, carry):
      pltpu.sync_copy(ones_v, acc.at[dst_v.at[j]], add=True)
      return carry

    lax.fori_loop(0, NCHUNK, body, 0)
    plsc.subcore_barrier()
    pltpu.sync_copy(
        acc.at[pl.ds(s * N_TILE, N_TILE)],
        out_hbm.at[c, pl.ds(s * N_TILE, N_TILE)],
    )

  return deg_kernel


def _make_prop(F, NBUF, dt):
  @functools.partial(
      pl.kernel,
      out_type=jax.ShapeDtypeStruct((NC, N_PAD, F), dt),
      mesh=_MESH,
      compiler_params=pltpu.CompilerParams(use_tc_tiling_on_sc=False),
      scratch_types=[
          pltpu.VMEM_SHARED((N_PAD, F), dt),
          pltpu.VMEM((NCHUNK, CHUNK), jnp.int32),
          pltpu.VMEM((NCHUNK, CHUNK), jnp.int32),
      ]
      + [pltpu.VMEM((CHUNK, F), dt) for _ in range(NBUF)]
      + [pltpu.SemaphoreType.DMA for _ in range(NBUF)],
  )
  def prop_kernel(ei_hbm, xn_hbm, zeros_hbm, out_hbm, acc, src_v, dst_v, *bufs):
    rows = bufs[:NBUF]
    sems = bufs[NBUF:]
    c = lax.axis_index("c")
    s = lax.axis_index("s")
    tid = c * NS + s
    pltpu.sync_copy(ei_hbm.at[0, tid], src_v)
    pltpu.sync_copy(ei_hbm.at[1, tid], dst_v)
    pltpu.sync_copy(zeros_hbm, acc.at[pl.ds(s * N_TILE, N_TILE)])
    plsc.subcore_barrier()

    def stage(j, b):
      pltpu.async_copy(xn_hbm.at[src_v.at[j]], rows[b], sems[b])

    def finish(j, b):
      pltpu.make_async_copy(xn_hbm.at[src_v.at[j]], rows[b], sems[b]).wait()
      pltpu.sync_copy(rows[b], acc.at[dst_v.at[j]], add=True)

    for b in range(NBUF):
      stage(b, b)

    def body(g, carry):
      base = g * NBUF
      for b in range(NBUF):
        finish(base + b, b)
        stage(base + NBUF + b, b)
      return carry

    lax.fori_loop(0, NCHUNK // NBUF - 1, body, 0)
    for b in range(NBUF):
      finish(NCHUNK - NBUF + b, b)
    plsc.subcore_barrier()
    pltpu.sync_copy(
        acc.at[pl.ds(s * N_TILE, N_TILE)],
        out_hbm.at[c, pl.ds(s * N_TILE, N_TILE)],
    )

  return prop_kernel


_deg = _make_deg()
_prop128 = _make_prop(IN_F, 2, jnp.bfloat16)
_prop64 = _make_prop(C_F, 4, jnp.bfloat16)

BR = 1000
GRID = N // BR


def _dinv_block(dacc_ref):
  i = pl.program_id(0)
  rows = pl.ds(i * BR, BR)
  d = dacc_ref[0, rows, 0:1] + dacc_ref[1, rows, 0:1] + 1.0
  return lax.rsqrt(d)


_DACC_SPEC = pl.BlockSpec((NC, N_PAD, 16), lambda i: (0, 0, 0))


def _tc_scale(dacc, x):
  def body(dacc_ref, x_ref, o_ref):
    o_ref[...] = (x_ref[...] * _dinv_block(dacc_ref)).astype(jnp.bfloat16)

  return pl.pallas_call(
      body,
      grid=(GRID,),
      in_specs=[
          _DACC_SPEC,
          pl.BlockSpec((BR, IN_F), lambda i: (i, 0)),
      ],
      out_specs=pl.BlockSpec((BR, IN_F), lambda i: (i, 0)),
      out_shape=jax.ShapeDtypeStruct((N, IN_F), jnp.bfloat16),
  )(dacc, x)


def _tc_mid(dacc, agg1, xn1, W1, b1, W2):
  def body(dacc_ref, a_ref, xn_ref, W1_ref, b1_ref, W2_ref, o_ref):
    dinv = _dinv_block(dacc_ref)
    p = (
        a_ref[0].astype(jnp.float32)
        + a_ref[1].astype(jnp.float32)
        + xn_ref[...].astype(jnp.float32)
    ) * dinv
    h = jnp.tanh(
        jnp.dot(p, W1_ref[...], preferred_element_type=jnp.float32)
        + b1_ref[...]
    )
    o_ref[...] = (
        jnp.dot(h, W2_ref[...], preferred_element_type=jnp.float32) * dinv
    ).astype(jnp.bfloat16)

  return pl.pallas_call(
      body,
      grid=(GRID,),
      in_specs=[
          _DACC_SPEC,
          pl.BlockSpec((NC, BR, IN_F), lambda i: (0, i, 0)),
          pl.BlockSpec((BR, IN_F), lambda i: (i, 0)),
          pl.BlockSpec((IN_F, H_F), lambda i: (0, 0)),
          pl.BlockSpec((1, H_F), lambda i: (0, 0)),
          pl.BlockSpec((H_F, C_F), lambda i: (0, 0)),
      ],
      out_specs=pl.BlockSpec((BR, C_F), lambda i: (i, 0)),
      out_shape=jax.ShapeDtypeStruct((N, C_F), jnp.bfloat16),
  )(dacc, agg1, xn1, W1, b1, W2)


def _tc_final(dacc, agg2, xn2, b2):
  def body(dacc_ref, a_ref, xn_ref, b2_ref, o_ref):
    dinv = _dinv_block(dacc_ref)
    o_ref[...] = (
        a_ref[0].astype(jnp.float32)
        + a_ref[1].astype(jnp.float32)
        + xn_ref[...].astype(jnp.float32)
    ) * dinv + b2_ref[...]

  return pl.pallas_call(
      body,
      grid=(GRID,),
      in_specs=[
          _DACC_SPEC,
          pl.BlockSpec((NC, BR, C_F), lambda i: (0, i, 0)),
          pl.BlockSpec((BR, C_F), lambda i: (i, 0)),
          pl.BlockSpec((1, C_F), lambda i: (0, 0)),
      ],
      out_specs=pl.BlockSpec((BR, C_F), lambda i: (i, 0)),
      out_shape=jax.ShapeDtypeStruct((N, C_F), jnp.float32),
  )(dacc, agg2, xn2, b2)


def kernel(edge_index, x, W1, b1, W2, b2):
  ei = edge_index.reshape(2, NW, NCHUNK, CHUNK)
  ones16 = jnp.ones((CHUNK, 16), jnp.float32)
  zdeg = jnp.zeros((N_TILE, 16), jnp.float32)
  z128 = jnp.zeros((N_TILE, IN_F), jnp.bfloat16)
  z64 = jnp.zeros((N_TILE, C_F), jnp.bfloat16)

  dacc = _deg(ei, ones16, zdeg)
  xn1 = _tc_scale(dacc, x)
  agg1 = _prop128(ei, xn1, z128)
  xn2 = _tc_mid(dacc, agg1, xn1, W1, b1.reshape(1, H_F), W2)
  agg2 = _prop64(ei, xn2, z64)
  return _tc_final(dacc, agg2, xn2, b2.reshape(1, C_F))
